# Initial kernel scaffold; baseline (speedup 1.0000x reference)
import jax
import jax.numpy as jnp
from jax.experimental import pallas as pl


def kernel(laplacian_indices, laplacian_values, inputs, weight, bias):
    raise NotImplementedError("write your pallas kernel here")


# speedup vs baseline: 2.3452x; 2.3452x over previous
import dataclasses
import functools

import jax
import jax.numpy as jnp
from jax import lax
from jax.experimental import pallas as pl
from jax.experimental.pallas import tpu as pltpu
from jax.experimental.pallas import tpu_sc as plsc

B, V, E, FIN, FOUT, K = 2, 10000, 160000, 128, 128, 5

NC, NS = 2, 16
EPT = E // NS
G = 80
NCH_E = EPT // G
RB = 80
NCH_R = V // RB
LANES = 16
FC = FIN // LANES


def _sc_cheb_body(x0_hbm, rows_hbm, cols_hbm, vals_hbm, chain_hbm,
                  accum, colv, rowv, valv, stag, tbuf, pbuf, zbuf):
    c = lax.axis_index("c")
    s = lax.axis_index("s")
    cV = c * V

    @pl.loop(0, RB)
    def _(r):
        for t in range(FC):
            zbuf[r, pl.ds(t * LANES, LANES)] = jnp.zeros((LANES,), jnp.float32)

    for k in range(1, K):
        src_is_x0 = (k == 1)
        src_base = (k - 1) * (B * V)
        coef = 1.0 if k == 1 else 2.0

        @pl.loop(s, NCH_R, step=NS)
        def _(j):
            pltpu.sync_copy(zbuf, accum.at[pl.ds(j * RB, RB)])

        plsc.subcore_barrier()

        ebase = s * EPT

        @pl.loop(0, NCH_E)
        def _(j):
            off = ebase + j * G
            pltpu.sync_copy(cols_hbm.at[pl.ds(off, G)], colv)
            pltpu.sync_copy(rows_hbm.at[pl.ds(off, G)], rowv)
            pltpu.sync_copy(vals_hbm.at[pl.ds(off, G)], valv)
            shift = cV if src_is_x0 else src_base + cV
            for t in range(G // LANES):
                sl = pl.ds(t * LANES, LANES)
                colv[sl] = colv[sl] + shift
            if src_is_x0:
                pltpu.sync_copy(x0_hbm.at[colv], stag)
            else:
                pltpu.sync_copy(chain_hbm.at[colv], stag)

            @pl.loop(0, G)
            def _(g):
                gsplat = jnp.full((LANES,), 0, jnp.int32) + g
                vscale = plsc.load_gather(valv, [gsplat]) * coef
                for t in range(FC):
                    sl = pl.ds(t * LANES, LANES)
                    stag[g, sl] = stag[g, sl] * vscale

            pltpu.sync_copy(stag, accum.at[rowv], add=True)

        plsc.subcore_barrier()

        prev_is_x0 = (k == 2)
        prev_base = (k - 2) * (B * V)

        @pl.loop(s, NCH_R, step=NS)
        def _(j):
            r0 = j * RB
            pltpu.sync_copy(accum.at[pl.ds(r0, RB)], tbuf)
            if k >= 2:
                if prev_is_x0:
                    pltpu.sync_copy(x0_hbm.at[pl.ds(cV + r0, RB)], pbuf)
                else:
                    pltpu.sync_copy(
                        chain_hbm.at[pl.ds(prev_base + cV + r0, RB)], pbuf)

                @pl.loop(0, RB)
                def _(r):
                    for t in range(FC):
                        sl = pl.ds(t * LANES, LANES)
                        tbuf[r, sl] = tbuf[r, sl] - pbuf[r, sl]

            pltpu.sync_copy(
                tbuf, chain_hbm.at[pl.ds(k * (B * V) + cV + r0, RB)])

        plsc.subcore_barrier()


def _sc_compiler_params():
    cp = pltpu.CompilerParams()
    if "needs_layout_passes" in pltpu.CompilerParams.__dataclass_fields__:
        cp = dataclasses.replace(cp, needs_layout_passes=False)
    return cp


@jax.jit
def _sc_cheb(x0, rows, cols, vals):
    kern = pl.kernel(
        _sc_cheb_body,
        compiler_params=_sc_compiler_params(),
        out_type=jax.ShapeDtypeStruct((K * B * V, FIN), jnp.float32),
        mesh=plsc.VectorSubcoreMesh(core_axis_name="c", subcore_axis_name="s"),
        scratch_types=[
            pltpu.VMEM_SHARED((V, FIN), jnp.float32),
            pltpu.VMEM((G,), jnp.int32),
            pltpu.VMEM((G,), jnp.int32),
            pltpu.VMEM((G,), jnp.float32),
            pltpu.VMEM((G, FIN), jnp.float32),
            pltpu.VMEM((RB, FIN), jnp.float32),
            pltpu.VMEM((RB, FIN), jnp.float32),
            pltpu.VMEM((RB, FIN), jnp.float32),
        ],
    )
    return kern(x0, rows, cols, vals)


RBLK = 400


def _tc_dense_body(x0_ref, chain_ref, w_ref, bias_ref, out_ref):
    acc = jax.lax.dot_general(
        x0_ref[...], w_ref[0],
        (((1,), (0,)), ((), ())), preferred_element_type=jnp.float32)
    for k in range(1, K):
        acc += jax.lax.dot_general(
            chain_ref[k - 1], w_ref[k],
            (((1,), (0,)), ((), ())), preferred_element_type=jnp.float32)
    out_ref[...] = acc + bias_ref[...]


@jax.jit
def _tc_dense(x0, chain, wp, bias2d):
    chain3 = chain.reshape(K, B * V, FIN)[1:]
    grid = (B * V // RBLK,)
    return pl.pallas_call(
        _tc_dense_body,
        grid=grid,
        in_specs=[
            pl.BlockSpec((RBLK, FIN), lambda i: (i, 0)),
            pl.BlockSpec((K - 1, RBLK, FIN), lambda i: (0, i, 0)),
            pl.BlockSpec((K, FIN, FOUT), lambda i: (0, 0, 0)),
            pl.BlockSpec((1, FOUT), lambda i: (0, 0)),
        ],
        out_specs=pl.BlockSpec((RBLK, FOUT), lambda i: (i, 0)),
        out_shape=jax.ShapeDtypeStruct((B * V, FOUT), jnp.float32),
    )(x0, chain3, wp, bias2d)


def kernel(laplacian_indices, laplacian_values, inputs, weight, bias):
    rows = laplacian_indices[0]
    cols = laplacian_indices[1]
    x0 = inputs.reshape(B * V, FIN)
    chain = _sc_cheb(x0, rows, cols, laplacian_values)
    wp = weight.reshape(K * FIN, FOUT).reshape(FIN, K, FOUT).transpose(1, 0, 2)
    out = _tc_dense(x0, chain, wp, bias.reshape(1, FOUT))
    return out.reshape(B, V, FOUT)


# speedup vs baseline: 4.8130x; 2.0523x over previous
import dataclasses
import functools

import jax
import jax.numpy as jnp
from jax import lax
from jax.experimental import pallas as pl
from jax.experimental.pallas import tpu as pltpu
from jax.experimental.pallas import tpu_sc as plsc

B, V, E, FIN, FOUT, K = 2, 10000, 160000, 128, 128, 5

NC, NS = 2, 16
EPT = E // NS
G = 80
NCH_E = EPT // G
RB = 80
NCH_R = V // RB
LANES = 16
FC = FIN // LANES
EG = G // LANES


def _sc_cheb_body(x0_hbm, rows_hbm, cols_hbm, vals_hbm, chain_hbm,
                  accum, colv, rowv, valv, rb0, rb1, stag0, stag1,
                  sg0, sg1, ss0, ss1):
    c = lax.axis_index("c")
    s = lax.axis_index("s")
    cV = c * V
    NV = B * V

    pltpu.sync_copy(cols_hbm.at[s], colv)
    pltpu.sync_copy(rows_hbm.at[s], rowv)
    pltpu.sync_copy(vals_hbm.at[s], valv)

    @pl.loop(0, EPT // LANES)
    def _(j):
        sl = pl.ds(j * LANES, LANES)
        colv[sl] = colv[sl] + cV

    @pl.loop(s, NCH_R, step=NS)
    def _(j):
        r0 = cV + j * RB
        pltpu.sync_copy(x0_hbm.at[pl.ds(r0, RB)], stag0)
        pltpu.sync_copy(stag0, chain_hbm.at[pl.ds(r0, RB)])

    plsc.subcore_barrier()

    stags = (stag0, stag1)
    rbs = (rb0, rb1)
    sgs = (sg0, sg1)
    sss = (ss0, ss1)

    def issue_gather(ci, b):
        pltpu.async_copy(chain_hbm.at[colv.at[pl.ds(ci * G, G)]],
                         stags[b], sgs[b])

    def wait_gather(ci, b):
        pltpu.make_async_copy(chain_hbm.at[colv.at[pl.ds(ci * G, G)]],
                              stags[b], sgs[b]).wait()

    def scale(ci, b):
        st = stags[b]

        @pl.loop(0, EG)
        def _(t):
            vv = valv[pl.ds(ci * G + t * LANES, LANES)]
            for i in range(LANES):
                e = t * LANES + i
                vs = vv[i]
                for f in range(FC):
                    sl = pl.ds(f * LANES, LANES)
                    st[e, sl] = st[e, sl] * vs

    def fill_rowbuf(ci, b):
        for t in range(EG):
            rbs[b][pl.ds(t * LANES, LANES)] = (
                rowv[pl.ds(ci * G + t * LANES, LANES)])

    def issue_scatter(ci, b):
        pltpu.async_copy(stags[b], accum.at[rbs[b]], sss[b], add=True)

    def wait_scatter(b):
        pltpu.make_async_copy(stags[b], accum.at[rbs[b]], sss[b]).wait()

    def process(ci, b, issue_next, wait_prev):
        wait_gather(ci, b)
        scale(ci, b)
        fill_rowbuf(ci, b)
        issue_scatter(ci, b)
        if issue_next:
            if wait_prev:
                wait_scatter(1 - b)
            issue_gather(ci + 1, 1 - b)

    @pl.loop(1, K)
    def _(k):
        @pl.when(k >= 2)
        def _():
            @pl.loop(0, EPT // LANES)
            def _(j):
                sl = pl.ds(j * LANES, LANES)
                colv[sl] = colv[sl] + NV

            @pl.when(k == 2)
            def _():
                @pl.loop(0, EPT // LANES)
                def _(j):
                    sl = pl.ds(j * LANES, LANES)
                    valv[sl] = valv[sl] * 2.0

        @pl.loop(0, RB)
        def _(r):
            for t in range(FC):
                stag0[r, pl.ds(t * LANES, LANES)] = jnp.zeros(
                    (LANES,), jnp.float32)

        @pl.loop(s, NCH_R, step=NS)
        def _(j):
            pltpu.sync_copy(stag0, accum.at[pl.ds(j * RB, RB)])

        plsc.subcore_barrier()

        issue_gather(0, 0)
        process(0, 0, True, False)

        @pl.loop(0, (NCH_E - 3) // 2)
        def _(t):
            ci = 1 + 2 * t
            process(ci, 1, True, True)
            process(ci + 1, 0, True, True)

        process(NCH_E - 2, 1, True, True)
        process(NCH_E - 1, 0, False, False)
        wait_scatter(1)
        wait_scatter(0)

        plsc.subcore_barrier()

        @pl.loop(s, NCH_R, step=NS)
        def _(j):
            r0 = j * RB
            pltpu.sync_copy(accum.at[pl.ds(r0, RB)], stag0)

            @pl.when(k >= 2)
            def _():
                pltpu.sync_copy(
                    chain_hbm.at[pl.ds((k - 2) * NV + cV + r0, RB)], stag1)

                @pl.loop(0, RB)
                def _(r):
                    for t in range(FC):
                        sl = pl.ds(t * LANES, LANES)
                        stag0[r, sl] = stag0[r, sl] - stag1[r, sl]

            pltpu.sync_copy(stag0, chain_hbm.at[pl.ds(k * NV + cV + r0, RB)])

        plsc.subcore_barrier()


def _sc_compiler_params():
    cp = pltpu.CompilerParams()
    if "needs_layout_passes" in pltpu.CompilerParams.__dataclass_fields__:
        cp = dataclasses.replace(cp, needs_layout_passes=False)
    return cp


@jax.jit
def _sc_cheb(x0, rows2, cols2, vals2):
    kern = pl.kernel(
        _sc_cheb_body,
        compiler_params=_sc_compiler_params(),
        out_type=jax.ShapeDtypeStruct((K * B * V, FIN), jnp.float32),
        mesh=plsc.VectorSubcoreMesh(core_axis_name="c", subcore_axis_name="s"),
        scratch_types=[
            pltpu.VMEM_SHARED((V, FIN), jnp.float32),
            pltpu.VMEM((EPT,), jnp.int32),
            pltpu.VMEM((EPT,), jnp.int32),
            pltpu.VMEM((EPT,), jnp.float32),
            pltpu.VMEM((G,), jnp.int32),
            pltpu.VMEM((G,), jnp.int32),
            pltpu.VMEM((G, FIN), jnp.float32),
            pltpu.VMEM((G, FIN), jnp.float32),
            pltpu.SemaphoreType.DMA,
            pltpu.SemaphoreType.DMA,
            pltpu.SemaphoreType.DMA,
            pltpu.SemaphoreType.DMA,
        ],
    )
    return kern(x0, rows2, cols2, vals2)


RBLK = 400


def _tc_dense_body(x0_ref, chain_ref, w_ref, bias_ref, out_ref):
    acc = jax.lax.dot_general(
        x0_ref[...], w_ref[0],
        (((1,), (0,)), ((), ())), preferred_element_type=jnp.float32)
    for k in range(1, K):
        acc += jax.lax.dot_general(
            chain_ref[k - 1], w_ref[k],
            (((1,), (0,)), ((), ())), preferred_element_type=jnp.float32)
    out_ref[...] = acc + bias_ref[...]


@jax.jit
def _tc_dense(x0, chain, wp, bias2d):
    chain3 = chain.reshape(K, B * V, FIN)[1:]
    grid = (B * V // RBLK,)
    return pl.pallas_call(
        _tc_dense_body,
        grid=grid,
        in_specs=[
            pl.BlockSpec((RBLK, FIN), lambda i: (i, 0)),
            pl.BlockSpec((K - 1, RBLK, FIN), lambda i: (0, i, 0)),
            pl.BlockSpec((K, FIN, FOUT), lambda i: (0, 0, 0)),
            pl.BlockSpec((1, FOUT), lambda i: (0, 0)),
        ],
        out_specs=pl.BlockSpec((RBLK, FOUT), lambda i: (i, 0)),
        out_shape=jax.ShapeDtypeStruct((B * V, FOUT), jnp.float32),
    )(x0, chain3, wp, bias2d)


def kernel(laplacian_indices, laplacian_values, inputs, weight, bias):
    rows2 = laplacian_indices[0].reshape(NS, EPT)
    cols2 = laplacian_indices[1].reshape(NS, EPT)
    vals2 = laplacian_values.reshape(NS, EPT)
    x0 = inputs.reshape(B * V, FIN)
    chain = _sc_cheb(x0, rows2, cols2, vals2)
    wp = weight.reshape(K * FIN, FOUT).reshape(FIN, K, FOUT).transpose(1, 0, 2)
    out = _tc_dense(x0, chain, wp, bias2d=bias.reshape(1, FOUT))
    return out.reshape(B, V, FOUT)


# speedup vs baseline: 6.0623x; 1.2596x over previous
import dataclasses
import functools

import jax
import jax.numpy as jnp
from jax import lax
from jax.experimental import pallas as pl
from jax.experimental.pallas import tpu as pltpu
from jax.experimental.pallas import tpu_sc as plsc

B, V, E, FIN, FOUT, K = 2, 10000, 160000, 128, 128, 5

NC, NS = 2, 16
EPT = E // NS
G = 80
NCH_E = EPT // G
RB = 80
NCH_R = V // RB
LANES = 16
FC = FIN // LANES
EG = G // LANES


def _sc_cheb_body(x0_hbm, rows_hbm, cols_hbm, vals_hbm, chain_hbm,
                  accum, colv, rowv, valv, rb0, rb1, stag0, stag1,
                  sg0, sg1, ss0, ss1):
    c = lax.axis_index("c")
    s = lax.axis_index("s")
    cV = c * V
    NV = B * V

    pltpu.sync_copy(cols_hbm.at[s], colv)
    pltpu.sync_copy(rows_hbm.at[s], rowv)
    pltpu.sync_copy(vals_hbm.at[s], valv)

    @pl.loop(0, EPT // LANES)
    def _(j):
        sl = pl.ds(j * LANES, LANES)
        colv[sl] = colv[sl] + cV

    @pl.loop(s, NCH_R, step=NS)
    def _(j):
        r0 = cV + j * RB
        pltpu.sync_copy(x0_hbm.at[pl.ds(r0, RB)], stag0)
        pltpu.sync_copy(stag0, chain_hbm.at[pl.ds(r0, RB)])

    plsc.subcore_barrier()

    stags = (stag0, stag1)
    rbs = (rb0, rb1)
    sgs = (sg0, sg1)
    sss = (ss0, ss1)

    def issue_gather(ci, b):
        pltpu.async_copy(chain_hbm.at[colv.at[pl.ds(ci * G, G)]],
                         stags[b], sgs[b])

    def wait_gather(ci, b):
        pltpu.make_async_copy(chain_hbm.at[colv.at[pl.ds(ci * G, G)]],
                              stags[b], sgs[b]).wait()

    def scale(ci, b):
        st = stags[b]

        @pl.loop(0, EG)
        def _(t):
            vv = valv[pl.ds(ci * G + t * LANES, LANES)]
            for i in range(LANES):
                e = t * LANES + i
                vs = vv[i]
                for f in range(FC):
                    sl = pl.ds(f * LANES, LANES)
                    st[e, sl] = st[e, sl] * vs

    def fill_rowbuf(ci, b):
        for t in range(EG):
            rbs[b][pl.ds(t * LANES, LANES)] = (
                rowv[pl.ds(ci * G + t * LANES, LANES)])

    def issue_scatter(ci, b):
        pltpu.async_copy(stags[b], accum.at[rbs[b]], sss[b], add=True)

    def wait_scatter(b):
        pltpu.make_async_copy(stags[b], accum.at[rbs[b]], sss[b]).wait()

    def process(ci, b, issue_next, wait_prev):
        wait_gather(ci, b)
        if issue_next:
            if wait_prev:
                wait_scatter(1 - b)
            issue_gather(ci + 1, 1 - b)
        scale(ci, b)
        fill_rowbuf(ci, b)
        issue_scatter(ci, b)

    @pl.loop(1, K)
    def _(k):
        @pl.when(k >= 2)
        def _():
            @pl.loop(0, EPT // LANES)
            def _(j):
                sl = pl.ds(j * LANES, LANES)
                colv[sl] = colv[sl] + NV

            @pl.when(k == 2)
            def _():
                @pl.loop(0, EPT // LANES)
                def _(j):
                    sl = pl.ds(j * LANES, LANES)
                    valv[sl] = valv[sl] * 2.0

        @pl.loop(0, RB)
        def _(r):
            for t in range(FC):
                stag0[r, pl.ds(t * LANES, LANES)] = jnp.zeros(
                    (LANES,), jnp.float32)

        @pl.loop(s, NCH_R, step=NS)
        def _(j):
            pltpu.sync_copy(stag0, accum.at[pl.ds(j * RB, RB)])

        plsc.subcore_barrier()

        issue_gather(0, 0)
        process(0, 0, True, False)

        @pl.loop(0, (NCH_E - 3) // 2)
        def _(t):
            ci = 1 + 2 * t
            process(ci, 1, True, True)
            process(ci + 1, 0, True, True)

        process(NCH_E - 2, 1, True, True)
        process(NCH_E - 1, 0, False, False)
        wait_scatter(1)
        wait_scatter(0)

        plsc.subcore_barrier()

        @pl.loop(s, NCH_R, step=NS)
        def _(j):
            r0 = j * RB
            pltpu.sync_copy(accum.at[pl.ds(r0, RB)], stag0)

            @pl.when(k >= 2)
            def _():
                pltpu.sync_copy(
                    chain_hbm.at[pl.ds((k - 2) * NV + cV + r0, RB)], stag1)

                @pl.loop(0, RB)
                def _(r):
                    for t in range(FC):
                        sl = pl.ds(t * LANES, LANES)
                        stag0[r, sl] = stag0[r, sl] - stag1[r, sl]

            pltpu.sync_copy(stag0, chain_hbm.at[pl.ds(k * NV + cV + r0, RB)])

        plsc.subcore_barrier()


def _sc_compiler_params():
    cp = pltpu.CompilerParams()
    if "needs_layout_passes" in pltpu.CompilerParams.__dataclass_fields__:
        cp = dataclasses.replace(cp, needs_layout_passes=False)
    return cp


@jax.jit
def _sc_cheb(x0, rows2, cols2, vals2):
    kern = pl.kernel(
        _sc_cheb_body,
        compiler_params=_sc_compiler_params(),
        out_type=jax.ShapeDtypeStruct((K * B * V, FIN), jnp.float32),
        mesh=plsc.VectorSubcoreMesh(core_axis_name="c", subcore_axis_name="s"),
        scratch_types=[
            pltpu.VMEM_SHARED((V, FIN), jnp.float32),
            pltpu.VMEM((EPT,), jnp.int32),
            pltpu.VMEM((EPT,), jnp.int32),
            pltpu.VMEM((EPT,), jnp.float32),
            pltpu.VMEM((G,), jnp.int32),
            pltpu.VMEM((G,), jnp.int32),
            pltpu.VMEM((G, FIN), jnp.float32),
            pltpu.VMEM((G, FIN), jnp.float32),
            pltpu.SemaphoreType.DMA,
            pltpu.SemaphoreType.DMA,
            pltpu.SemaphoreType.DMA,
            pltpu.SemaphoreType.DMA,
        ],
    )
    return kern(x0, rows2, cols2, vals2)


RBLK = 400


def _tc_dense_body(x0_ref, chain_ref, w_ref, bias_ref, out_ref):
    acc = jax.lax.dot_general(
        x0_ref[...], w_ref[0],
        (((1,), (0,)), ((), ())), preferred_element_type=jnp.float32)
    for k in range(1, K):
        acc += jax.lax.dot_general(
            chain_ref[k - 1], w_ref[k],
            (((1,), (0,)), ((), ())), preferred_element_type=jnp.float32)
    out_ref[...] = acc + bias_ref[...]


@jax.jit
def _tc_dense(x0, chain, wp, bias2d):
    chain3 = chain.reshape(K, B * V, FIN)[1:]
    grid = (B * V // RBLK,)
    return pl.pallas_call(
        _tc_dense_body,
        grid=grid,
        in_specs=[
            pl.BlockSpec((RBLK, FIN), lambda i: (i, 0)),
            pl.BlockSpec((K - 1, RBLK, FIN), lambda i: (0, i, 0)),
            pl.BlockSpec((K, FIN, FOUT), lambda i: (0, 0, 0)),
            pl.BlockSpec((1, FOUT), lambda i: (0, 0)),
        ],
        out_specs=pl.BlockSpec((RBLK, FOUT), lambda i: (i, 0)),
        out_shape=jax.ShapeDtypeStruct((B * V, FOUT), jnp.float32),
    )(x0, chain3, wp, bias2d)


def kernel(laplacian_indices, laplacian_values, inputs, weight, bias):
    rows2 = laplacian_indices[0].reshape(NS, EPT)
    cols2 = laplacian_indices[1].reshape(NS, EPT)
    vals2 = laplacian_values.reshape(NS, EPT)
    x0 = inputs.reshape(B * V, FIN)
    chain = _sc_cheb(x0, rows2, cols2, vals2)
    wp = weight.reshape(K * FIN, FOUT).reshape(FIN, K, FOUT).transpose(1, 0, 2)
    out = _tc_dense(x0, chain, wp, bias2d=bias.reshape(1, FOUT))
    return out.reshape(B, V, FOUT)
